# Initial kernel scaffold; baseline (speedup 1.0000x reference)
import jax
import jax.numpy as jnp
from jax.experimental import pallas as pl


def kernel(x, edge_index, edge_attr, batch, u, W1, b1, W2, b2, Wm1, bm1, Wm2, bm2):
    raise NotImplementedError("write your pallas kernel here")


# speedup vs baseline: 15.1019x; 15.1019x over previous
import functools

import jax
import jax.numpy as jnp
from jax import lax
from jax.experimental import pallas as pl
from jax.experimental.pallas import tpu as pltpu
from jax.experimental.pallas import tpu_sc as plsc

N = 10000
E = 320000
D = 128
H = 128
G = 32
B = 16

N_PAD = 10240
K = 80
EC = E // K
NC = 2
NS = 16
NW = NC * NS
CPT = EC // NW
GRP = 25
NGRP = CPT // GRP
RPT = N_PAD // NS

_mesh = plsc.VectorSubcoreMesh(core_axis_name="c", subcore_axis_name="s")


@functools.partial(
    pl.kernel,
    out_type=(
        jax.ShapeDtypeStruct((NW, CPT, K), jnp.float32),
        jax.ShapeDtypeStruct((N_PAD,), jnp.float32),
        jax.ShapeDtypeStruct((N_PAD,), jnp.float32),
    ),
    mesh=_mesh,
    scratch_types=[
        pltpu.VMEM((CPT, K), jnp.float32),
        pltpu.VMEM((CPT, K), jnp.float32),
        pltpu.VMEM((CPT, K), jnp.int32),
        pltpu.VMEM((RPT,), jnp.float32),
        pltpu.VMEM_SHARED((N_PAD,), jnp.float32),
    ],
)
def _sc_ew_deg(d_hbm, dst_hbm, ew_hbm, deg0_hbm, deg1_hbm,
               d_v, ew_v, dst_v, zero_v, deg_sh):
    c = lax.axis_index("c")
    s = lax.axis_index("s")
    w = s * NC + c

    def zfill(i, _):
        zero_v[pl.ds(i * 16, 16)] = jnp.zeros((16,), jnp.float32)
        return 0

    lax.fori_loop(0, RPT // 16, zfill, 0)
    pltpu.sync_copy(zero_v, deg_sh.at[pl.ds(s * RPT, RPT)])

    pltpu.sync_copy(d_hbm.at[w], d_v)
    pltpu.sync_copy(dst_hbm.at[w], dst_v)

    def cw(i, _):
        for j in range(K // 16):
            sl = pl.ds(j * 16, 16)
            d = d_v[i, sl]
            t = d * 0.2
            ew_v[i, sl] = jnp.exp(-(t * t))
        return 0

    lax.fori_loop(0, CPT, cw, 0)
    pltpu.sync_copy(ew_v, ew_hbm.at[w])

    plsc.subcore_barrier()

    def sa(i, _):
        pltpu.sync_copy(ew_v.at[i], deg_sh.at[dst_v.at[i]], add=True)
        return 0

    lax.fori_loop(0, CPT, sa, 0)
    plsc.subcore_barrier()

    @pl.when(c == 0)
    def _():
        pltpu.sync_copy(deg_sh.at[pl.ds(s * RPT, RPT)],
                        deg0_hbm.at[pl.ds(s * RPT, RPT)])

    @pl.when(c == 1)
    def _():
        pltpu.sync_copy(deg_sh.at[pl.ds(s * RPT, RPT)],
                        deg1_hbm.at[pl.ds(s * RPT, RPT)])


@functools.partial(
    pl.kernel,
    out_type=jax.ShapeDtypeStruct((NC, N_PAD, D), jnp.float32),
    mesh=_mesh,
    scratch_types=[
        pltpu.VMEM((GRP, K), jnp.int32),
        pltpu.VMEM((GRP, K), jnp.int32),
        pltpu.VMEM((GRP, K), jnp.float32),
        pltpu.VMEM((K, D), jnp.float32),
        pltpu.VMEM((16, D), jnp.float32),
        pltpu.VMEM_SHARED((N_PAD, D), jnp.float32),
        pltpu.SemaphoreType.DMA,
    ],
)
def _sc_spmm(y_hbm, src_hbm, dst_hbm, ew_hbm, out_hbm,
             src_v, dst_v, ew_v, rows_v, zrow_v, acc_sh, sem):
    c = lax.axis_index("c")
    s = lax.axis_index("s")
    w = s * NC + c

    def zfill(i, _):
        for j in range(D // 16):
            zrow_v[i, pl.ds(j * 16, 16)] = jnp.zeros((16,), jnp.float32)
        return 0

    lax.fori_loop(0, 16, zfill, 0)

    def zcopy(i, _):
        pltpu.sync_copy(zrow_v, acc_sh.at[pl.ds(s * RPT + i * 16, 16)])
        return 0

    lax.fori_loop(0, RPT // 16, zcopy, 0)
    plsc.subcore_barrier()

    def group_fn(gi, _):
        pltpu.sync_copy(src_hbm.at[w, gi], src_v)
        pltpu.sync_copy(dst_hbm.at[w, gi], dst_v)
        pltpu.sync_copy(ew_hbm.at[w, gi], ew_v)

        def step(i, _):
            pltpu.async_copy(y_hbm.at[src_v.at[i]], rows_v, sem).wait()
            for g in range(K // 16):
                cvec = ew_v[i, pl.ds(g * 16, 16)]
                for k in range(16):
                    cf = jnp.broadcast_to(cvec[k:k + 1], (16,))
                    r = g * 16 + k
                    for j in range(D // 16):
                        sl = pl.ds(j * 16, 16)
                        rows_v[r, sl] = rows_v[r, sl] * cf
            pltpu.sync_copy(rows_v, acc_sh.at[dst_v.at[i]], add=True)
            return 0

        lax.fori_loop(0, GRP, step, 0)
        return 0

    lax.fori_loop(0, NGRP, group_fn, 0)

    plsc.subcore_barrier()
    pltpu.sync_copy(acc_sh.at[pl.ds(s * RPT, RPT)],
                    out_hbm.at[c, pl.ds(s * RPT, RPT)])


BK = 1024
NBLK = N_PAD // BK


def _tc_dinv_y0_body(deg0_ref, deg1_ref, x_ref, dinv_ref, y0_ref):
    deg = 1.0 + deg0_ref[...] + deg1_ref[...]
    dinv = lax.rsqrt(deg)
    dinv_ref[...] = dinv
    y0_ref[...] = dinv * x_ref[...]


def _tc_layer_body(p0_ref, p1_ref, y_ref, dinv_ref, wt_ref, b_ref, yout_ref):
    agg = (p0_ref[...] + p1_ref[...] + y_ref[...]) * dinv_ref[...]
    h = jnp.maximum(
        jax.lax.dot(agg, wt_ref[...], precision=lax.Precision.HIGHEST,
                    preferred_element_type=jnp.float32) + b_ref[...],
        0.0)
    yout_ref[...] = h * dinv_ref[...]


def _tc_final_body(p0_ref, p1_ref, y_ref, dinv_ref, batch_ref,
                   w2t_ref, b2_ref, u_ref, wm1g_ref, wm1u_ref, bm1_ref,
                   wm2_ref, bm2_ref, out_ref, s_acc, cnt_acc):
    i = pl.program_id(0)

    @pl.when(i == 0)
    def _():
        s_acc[...] = jnp.zeros_like(s_acc)
        cnt_acc[...] = jnp.zeros_like(cnt_acc)

    agg = (p0_ref[...] + p1_ref[...] + y_ref[...]) * dinv_ref[...]
    t2 = jnp.maximum(
        jax.lax.dot(agg, w2t_ref[...], precision=lax.Precision.HIGHEST,
                    preferred_element_type=jnp.float32) + b2_ref[...],
        0.0)
    segid = lax.broadcasted_iota(jnp.int32, (BK, B), 1)
    onehot = (batch_ref[...] == segid).astype(jnp.float32)
    dn = (((0,), (0,)), ((), ()))
    s_acc[...] += lax.dot_general(onehot, t2, dn,
                                  precision=lax.Precision.HIGHEST,
                                  preferred_element_type=jnp.float32)
    cnt_acc[...] += lax.dot_general(onehot, jnp.ones((BK, 1), jnp.float32), dn,
                                    precision=lax.Precision.HIGHEST,
                                    preferred_element_type=jnp.float32)

    @pl.when(i == NBLK - 1)
    def _():
        g = s_acc[...] / jnp.maximum(cnt_acc[...], 1.0)
        z1 = jnp.maximum(
            jax.lax.dot(g, wm1g_ref[...], precision=lax.Precision.HIGHEST,
                        preferred_element_type=jnp.float32)
            + jax.lax.dot(u_ref[...], wm1u_ref[...],
                          precision=lax.Precision.HIGHEST,
                          preferred_element_type=jnp.float32)
            + bm1_ref[...],
            0.0)
        out_ref[...] = (jax.lax.dot(z1, wm2_ref[...],
                                    precision=lax.Precision.HIGHEST,
                                    preferred_element_type=jnp.float32)
                        + bm2_ref[...])


def _rows_spec(cols):
    return pl.BlockSpec((BK, cols), lambda i: (i, 0))


def _full_spec(shape):
    return pl.BlockSpec(shape, lambda i: tuple(0 for _ in shape))


def kernel(x, edge_index, edge_attr, batch, u, W1, b1, W2, b2, Wm1, bm1, Wm2, bm2):
    f32 = jnp.float32
    x = x.astype(f32)
    x_pad = jnp.pad(x, ((0, N_PAD - N), (0, 0)))
    d2 = edge_attr.reshape(NW, CPT, K).astype(f32)
    src2 = edge_index[0].reshape(NW, CPT, K)
    dst2 = edge_index[1].reshape(NW, CPT, K)
    batch_pad = jnp.pad(batch, (0, N_PAD - N), constant_values=B).reshape(N_PAD, 1)
    W1t = W1.T.astype(f32)
    W2t = W2.T.astype(f32)
    Wm1t = Wm1.T.astype(f32)
    Wm1g = Wm1t[:H]
    Wm1u = Wm1t[H:]
    b1r = b1.reshape(1, H).astype(f32)
    b2r = b2.reshape(1, H).astype(f32)
    bm1r = bm1.reshape(1, 128).astype(f32)
    Wm2t = Wm2.T.astype(f32)
    bm2r = bm2.reshape(1, 1).astype(f32)

    ew2, dga, dgb = _sc_ew_deg(d2, dst2)
    deg0 = dga.reshape(N_PAD, 1)
    deg1 = dgb.reshape(N_PAD, 1)
    ew4 = ew2.reshape(NW, NGRP, GRP, K)
    src4 = src2.reshape(NW, NGRP, GRP, K)
    dst4 = dst2.reshape(NW, NGRP, GRP, K)

    dinv, y0 = pl.pallas_call(
        _tc_dinv_y0_body,
        grid=(NBLK,),
        in_specs=[_rows_spec(1), _rows_spec(1), _rows_spec(D)],
        out_specs=[_rows_spec(1), _rows_spec(D)],
        out_shape=[
            jax.ShapeDtypeStruct((N_PAD, 1), f32),
            jax.ShapeDtypeStruct((N_PAD, D), f32),
        ],
    )(deg0, deg1, x_pad)

    p = _sc_spmm(y0, src4, dst4, ew4)

    y1 = pl.pallas_call(
        _tc_layer_body,
        grid=(NBLK,),
        in_specs=[_rows_spec(D), _rows_spec(D), _rows_spec(D), _rows_spec(1),
                  _full_spec((D, H)), _full_spec((1, H))],
        out_specs=_rows_spec(H),
        out_shape=jax.ShapeDtypeStruct((N_PAD, H), f32),
    )(p[0], p[1], y0, dinv, W1t, b1r)

    q = _sc_spmm(y1, src4, dst4, ew4)

    out = pl.pallas_call(
        _tc_final_body,
        grid=(NBLK,),
        in_specs=[_rows_spec(H), _rows_spec(H), _rows_spec(H), _rows_spec(1),
                  _rows_spec(1),
                  _full_spec((H, H)), _full_spec((1, H)), _full_spec((B, G)),
                  _full_spec((H, 128)), _full_spec((G, 128)),
                  _full_spec((1, 128)), _full_spec((128, 1)),
                  _full_spec((1, 1))],
        out_specs=pl.BlockSpec((B, 1), lambda i: (0, 0)),
        out_shape=jax.ShapeDtypeStruct((B, 1), f32),
        scratch_shapes=[
            pltpu.VMEM((B, H), f32),
            pltpu.VMEM((B, 1), f32),
        ],
    )(q[0], q[1], y1, dinv, batch_pad,
      W2t, b2r, u, Wm1g, Wm1u, bm1r, Wm2t, bm2r)

    return out.reshape(-1)


# speedup vs baseline: 23.0100x; 1.5237x over previous
import functools

import jax
import jax.numpy as jnp
from jax import lax
from jax.experimental import pallas as pl
from jax.experimental.pallas import tpu as pltpu
from jax.experimental.pallas import tpu_sc as plsc

N = 10000
E = 320000
D = 128
H = 128
G = 32
B = 16

N_PAD = 10240
NC = 2
NS = 16
NW = NC * NS
RPT = N_PAD // NS

K = 80
CPT = (E // K) // NW

KS = 32
PKR = 80
EPT = PKR * 128
EPAD = EPT - E // NW

_mesh = plsc.VectorSubcoreMesh(core_axis_name="c", subcore_axis_name="s")


@functools.partial(
    pl.kernel,
    out_type=(
        jax.ShapeDtypeStruct((NW, CPT, K), jnp.float32),
        jax.ShapeDtypeStruct((N_PAD,), jnp.float32),
        jax.ShapeDtypeStruct((N_PAD,), jnp.float32),
    ),
    mesh=_mesh,
    scratch_types=[
        pltpu.VMEM((CPT, K), jnp.float32),
        pltpu.VMEM((CPT, K), jnp.float32),
        pltpu.VMEM((CPT, K), jnp.int32),
        pltpu.VMEM((RPT,), jnp.float32),
        pltpu.VMEM_SHARED((N_PAD,), jnp.float32),
    ],
)
def _sc_ew_deg(d_hbm, dst_hbm, ew_hbm, deg0_hbm, deg1_hbm,
               d_v, ew_v, dst_v, zero_v, deg_sh):
    c = lax.axis_index("c")
    s = lax.axis_index("s")
    w = s * NC + c

    def zfill(i, _):
        zero_v[pl.ds(i * 16, 16)] = jnp.zeros((16,), jnp.float32)
        return 0

    lax.fori_loop(0, RPT // 16, zfill, 0)
    pltpu.sync_copy(zero_v, deg_sh.at[pl.ds(s * RPT, RPT)])

    pltpu.sync_copy(d_hbm.at[w], d_v)
    pltpu.sync_copy(dst_hbm.at[w], dst_v)

    def cw(i, _):
        for j in range(K // 16):
            sl = pl.ds(j * 16, 16)
            d = d_v[i, sl]
            t = d * 0.2
            ew_v[i, sl] = jnp.exp(-(t * t))
        return 0

    lax.fori_loop(0, CPT, cw, 0)
    pltpu.sync_copy(ew_v, ew_hbm.at[w])

    plsc.subcore_barrier()

    def sa(i, _):
        pltpu.sync_copy(ew_v.at[i], deg_sh.at[dst_v.at[i]], add=True)
        return 0

    lax.fori_loop(0, CPT, sa, 0)
    plsc.subcore_barrier()

    @pl.when(c == 0)
    def _():
        pltpu.sync_copy(deg_sh.at[pl.ds(s * RPT, RPT)],
                        deg0_hbm.at[pl.ds(s * RPT, RPT)])

    @pl.when(c == 1)
    def _():
        pltpu.sync_copy(deg_sh.at[pl.ds(s * RPT, RPT)],
                        deg1_hbm.at[pl.ds(s * RPT, RPT)])


@functools.partial(
    pl.kernel,
    out_type=jax.ShapeDtypeStruct((NC, N_PAD, D), jnp.float32),
    mesh=_mesh,
    scratch_types=[
        pltpu.VMEM((PKR, 128), jnp.int32),
        pltpu.VMEM((PKR, 128), jnp.float32),
        pltpu.VMEM((3, KS), jnp.int32),
        pltpu.VMEM((3, KS), jnp.int32),
        pltpu.VMEM((KS, D), jnp.float32),
        pltpu.VMEM((KS, D), jnp.float32),
        pltpu.VMEM((KS, D), jnp.float32),
        pltpu.VMEM((KS, D), jnp.float32),
        pltpu.VMEM((KS, D), jnp.float32),
        pltpu.VMEM((KS, D), jnp.float32),
        pltpu.VMEM_SHARED((N_PAD, D), jnp.float32),
        pltpu.SemaphoreType.DMA,
        pltpu.SemaphoreType.DMA,
        pltpu.SemaphoreType.DMA,
        pltpu.SemaphoreType.DMA,
        pltpu.SemaphoreType.DMA,
        pltpu.SemaphoreType.DMA,
    ],
)
def _sc_spmm(y_hbm, pk_hbm, ew_hbm, out_hbm,
             pk_v, ew_v, sidx, didx, grow0, grow1, grow2,
             sbuf0, sbuf1, sbuf2, acc_sh,
             gs0, gs1, gs2, ss0, ss1, ss2):
    c = lax.axis_index("c")
    s = lax.axis_index("s")
    w = s * NC + c
    grows = (grow0, grow1, grow2)
    sbufs = (sbuf0, sbuf1, sbuf2)
    gsem = (gs0, gs1, gs2)
    ssem = (ss0, ss1, ss2)
    NCH = EPT // KS

    def zfill(i, _):
        for j in range(D // 16):
            sbuf0[i, pl.ds(j * 16, 16)] = jnp.zeros((16,), jnp.float32)
        return 0

    lax.fori_loop(0, KS, zfill, 0)

    def zcopy(i, _):
        pltpu.sync_copy(sbuf0, acc_sh.at[pl.ds(s * RPT + i * KS, KS)])
        return 0

    lax.fori_loop(0, RPT // KS, zcopy, 0)

    pltpu.sync_copy(pk_hbm.at[w], pk_v)
    pltpu.sync_copy(ew_hbm.at[w], ew_v)
    plsc.subcore_barrier()

    def unpack(t, u):
        r = lax.shift_right_logical(t, 2)
        q = t & 3
        for j in range(KS // 16):
            v = pk_v[r, pl.ds(q * KS + j * 16, 16)]
            sl = pl.ds(j * 16, 16)
            sidx[u, sl] = v & 0xFFFF
            didx[u, sl] = lax.shift_right_logical(v, 16)

    def gather(u):
        pltpu.async_copy(y_hbm.at[sidx.at[u]], grows[u], gsem[u])

    def wait_gather(u):
        pltpu.make_async_copy(y_hbm.at[sidx.at[u]], grows[u], gsem[u]).wait()

    def scatter(u):
        pltpu.async_copy(sbufs[u], acc_sh.at[didx.at[u]], ssem[u], add=True)

    def wait_scatter(u):
        pltpu.make_async_copy(sbufs[u], acc_sh.at[didx.at[u]], ssem[u]).wait()

    def compute(t, u):
        r = lax.shift_right_logical(t, 2)
        q = t & 3

        def grp(g, _):
            cvec = ew_v[r, pl.ds(q * KS + g * 16, 16)]
            for k in range(16):
                cf = jnp.broadcast_to(cvec[k:k + 1], (16,))
                rr = g * 16 + k
                for j in range(D // 16):
                    sl = pl.ds(j * 16, 16)
                    sbufs[u][rr, sl] = grows[u][rr, sl] * cf
            return 0

        lax.fori_loop(0, KS // 16, grp, 0)
        scatter(u)

    unpack(0, 0)
    gather(0)
    unpack(1, 1)
    gather(1)

    def tri(g, _):
        for u in range(3):
            t = 3 * g + u
            tp = t + 2
            up = (u + 2) % 3

            @pl.when(tp < NCH)
            def _():
                @pl.when(tp >= 3)
                def _():
                    wait_scatter(up)

                unpack(tp, up)
                gather(up)

            wait_gather(u)
            compute(t, u)
        return 0

    lax.fori_loop(0, NCH // 3, tri, 0)
    wait_gather(0)
    compute(NCH - 2, 0)
    wait_gather(1)
    compute(NCH - 1, 1)

    wait_scatter(0)
    wait_scatter(1)
    wait_scatter(2)
    plsc.subcore_barrier()
    pltpu.sync_copy(acc_sh.at[pl.ds(s * RPT, RPT)],
                    out_hbm.at[c, pl.ds(s * RPT, RPT)])


BK = 1024
NBLK = N_PAD // BK


def _tc_dinv_y0_body(deg0_ref, deg1_ref, x_ref, dinv_ref, y0_ref):
    deg = 1.0 + deg0_ref[...] + deg1_ref[...]
    dinv = lax.rsqrt(deg)
    dinv_ref[...] = dinv
    y0_ref[...] = dinv * x_ref[...]


def _tc_layer_body(p0_ref, p1_ref, y_ref, dinv_ref, wt_ref, b_ref, yout_ref):
    agg = (p0_ref[...] + p1_ref[...] + y_ref[...]) * dinv_ref[...]
    h = jnp.maximum(
        jax.lax.dot(agg, wt_ref[...], precision=lax.Precision.HIGHEST,
                    preferred_element_type=jnp.float32) + b_ref[...],
        0.0)
    yout_ref[...] = h * dinv_ref[...]


def _tc_final_body(p0_ref, p1_ref, y_ref, dinv_ref, batch_ref,
                   w2t_ref, b2_ref, u_ref, wm1g_ref, wm1u_ref, bm1_ref,
                   wm2_ref, bm2_ref, out_ref, s_acc, cnt_acc):
    i = pl.program_id(0)

    @pl.when(i == 0)
    def _():
        s_acc[...] = jnp.zeros_like(s_acc)
        cnt_acc[...] = jnp.zeros_like(cnt_acc)

    agg = (p0_ref[...] + p1_ref[...] + y_ref[...]) * dinv_ref[...]
    t2 = jnp.maximum(
        jax.lax.dot(agg, w2t_ref[...], precision=lax.Precision.HIGHEST,
                    preferred_element_type=jnp.float32) + b2_ref[...],
        0.0)
    segid = lax.broadcasted_iota(jnp.int32, (BK, B), 1)
    onehot = (batch_ref[...] == segid).astype(jnp.float32)
    dn = (((0,), (0,)), ((), ()))
    s_acc[...] += lax.dot_general(onehot, t2, dn,
                                  precision=lax.Precision.HIGHEST,
                                  preferred_element_type=jnp.float32)
    cnt_acc[...] += lax.dot_general(onehot, jnp.ones((BK, 1), jnp.float32), dn,
                                    precision=lax.Precision.HIGHEST,
                                    preferred_element_type=jnp.float32)

    @pl.when(i == NBLK - 1)
    def _():
        g = s_acc[...] / jnp.maximum(cnt_acc[...], 1.0)
        z1 = jnp.maximum(
            jax.lax.dot(g, wm1g_ref[...], precision=lax.Precision.HIGHEST,
                        preferred_element_type=jnp.float32)
            + jax.lax.dot(u_ref[...], wm1u_ref[...],
                          precision=lax.Precision.HIGHEST,
                          preferred_element_type=jnp.float32)
            + bm1_ref[...],
            0.0)
        out_ref[...] = (jax.lax.dot(z1, wm2_ref[...],
                                    precision=lax.Precision.HIGHEST,
                                    preferred_element_type=jnp.float32)
                        + bm2_ref[...])


def _rows_spec(cols):
    return pl.BlockSpec((BK, cols), lambda i: (i, 0))


def _full_spec(shape):
    return pl.BlockSpec(shape, lambda i: tuple(0 for _ in shape))


def kernel(x, edge_index, edge_attr, batch, u, W1, b1, W2, b2, Wm1, bm1, Wm2, bm2):
    f32 = jnp.float32
    x = x.astype(f32)
    x_pad = jnp.pad(x, ((0, N_PAD - N), (0, 0)))
    d2 = edge_attr.reshape(NW, CPT, K).astype(f32)
    src2 = edge_index[0].reshape(NW, CPT, K)
    dst2 = edge_index[1].reshape(NW, CPT, K)
    batch_pad = jnp.pad(batch, (0, N_PAD - N), constant_values=B).reshape(N_PAD, 1)
    W1t = W1.T.astype(f32)
    W2t = W2.T.astype(f32)
    Wm1t = Wm1.T.astype(f32)
    Wm1g = Wm1t[:H]
    Wm1u = Wm1t[H:]
    b1r = b1.reshape(1, H).astype(f32)
    b2r = b2.reshape(1, H).astype(f32)
    bm1r = bm1.reshape(1, 128).astype(f32)
    Wm2t = Wm2.T.astype(f32)
    bm2r = bm2.reshape(1, 1).astype(f32)

    ept_raw = E // NW
    srcw = edge_index[0].reshape(NW, ept_raw)
    dstw = edge_index[1].reshape(NW, ept_raw)
    pad_src = jnp.broadcast_to(jnp.arange(EPAD, dtype=jnp.int32)[None, :],
                               (NW, EPAD))
    pad_dst = pad_src + N
    pkw = jnp.concatenate([srcw | (dstw << 16),
                           pad_src | (pad_dst << 16)], axis=1)
    pkw = pkw.reshape(NW, PKR, 128)

    ew2, dga, dgb = _sc_ew_deg(d2, dst2)
    deg0 = dga.reshape(N_PAD, 1)
    deg1 = dgb.reshape(N_PAD, 1)
    ew_sp = jnp.concatenate([ew2.reshape(NW, ept_raw),
                             jnp.zeros((NW, EPAD), jnp.float32)], axis=1)
    ew_sp = ew_sp.reshape(NW, PKR, 128)

    dinv, y0 = pl.pallas_call(
        _tc_dinv_y0_body,
        grid=(NBLK,),
        in_specs=[_rows_spec(1), _rows_spec(1), _rows_spec(D)],
        out_specs=[_rows_spec(1), _rows_spec(D)],
        out_shape=[
            jax.ShapeDtypeStruct((N_PAD, 1), f32),
            jax.ShapeDtypeStruct((N_PAD, D), f32),
        ],
    )(deg0, deg1, x_pad)

    p = _sc_spmm(y0, pkw, ew_sp)

    y1 = pl.pallas_call(
        _tc_layer_body,
        grid=(NBLK,),
        in_specs=[_rows_spec(D), _rows_spec(D), _rows_spec(D),
                  _rows_spec(1),
                  _full_spec((D, H)), _full_spec((1, H))],
        out_specs=_rows_spec(H),
        out_shape=jax.ShapeDtypeStruct((N_PAD, H), f32),
    )(p[0], p[1], y0, dinv, W1t, b1r)

    q = _sc_spmm(y1, pkw, ew_sp)

    out = pl.pallas_call(
        _tc_final_body,
        grid=(NBLK,),
        in_specs=[_rows_spec(H), _rows_spec(H), _rows_spec(H),
                  _rows_spec(1), _rows_spec(1),
                  _full_spec((H, H)), _full_spec((1, H)), _full_spec((B, G)),
                  _full_spec((H, 128)), _full_spec((G, 128)),
                  _full_spec((1, 128)), _full_spec((128, 1)),
                  _full_spec((1, 1))],
        out_specs=pl.BlockSpec((B, 1), lambda i: (0, 0)),
        out_shape=jax.ShapeDtypeStruct((B, 1), f32),
        scratch_shapes=[
            pltpu.VMEM((B, H), f32),
            pltpu.VMEM((B, 1), f32),
        ],
    )(q[0], q[1], y1, dinv, batch_pad,
      W2t, b2r, u, Wm1g, Wm1u, bm1r, Wm2t, bm2r)

    return out.reshape(-1)


# speedup vs baseline: 25.3371x; 1.1011x over previous
import functools

import jax
import jax.numpy as jnp
from jax import lax
from jax.experimental import pallas as pl
from jax.experimental.pallas import tpu as pltpu
from jax.experimental.pallas import tpu_sc as plsc

N = 10000
E = 320000
D = 128
H = 128
G = 32
B = 16

N_PAD = 10240
NC = 2
NS = 16
NW = NC * NS
RPT = N_PAD // NS

K = 80
CPT = (E // K) // NW

KS = 64
PKR = 80
EPT = PKR * 128
EPAD = EPT - E // NW

_mesh = plsc.VectorSubcoreMesh(core_axis_name="c", subcore_axis_name="s")


@functools.partial(
    pl.kernel,
    out_type=(
        jax.ShapeDtypeStruct((NW, CPT, K), jnp.float32),
        jax.ShapeDtypeStruct((N_PAD,), jnp.float32),
        jax.ShapeDtypeStruct((N_PAD,), jnp.float32),
    ),
    mesh=_mesh,
    scratch_types=[
        pltpu.VMEM((CPT, K), jnp.float32),
        pltpu.VMEM((CPT, K), jnp.float32),
        pltpu.VMEM((CPT, K), jnp.int32),
        pltpu.VMEM((RPT,), jnp.float32),
        pltpu.VMEM_SHARED((N_PAD,), jnp.float32),
    ],
)
def _sc_ew_deg(d_hbm, dst_hbm, ew_hbm, deg0_hbm, deg1_hbm,
               d_v, ew_v, dst_v, zero_v, deg_sh):
    c = lax.axis_index("c")
    s = lax.axis_index("s")
    w = s * NC + c

    def zfill(i, _):
        zero_v[pl.ds(i * 16, 16)] = jnp.zeros((16,), jnp.float32)
        return 0

    lax.fori_loop(0, RPT // 16, zfill, 0)
    pltpu.sync_copy(zero_v, deg_sh.at[pl.ds(s * RPT, RPT)])

    pltpu.sync_copy(d_hbm.at[w], d_v)
    pltpu.sync_copy(dst_hbm.at[w], dst_v)

    def cw(i, _):
        for j in range(K // 16):
            sl = pl.ds(j * 16, 16)
            d = d_v[i, sl]
            t = d * 0.2
            ew_v[i, sl] = jnp.exp(-(t * t))
        return 0

    lax.fori_loop(0, CPT, cw, 0)
    pltpu.sync_copy(ew_v, ew_hbm.at[w])

    plsc.subcore_barrier()

    def sa(i, _):
        pltpu.sync_copy(ew_v.at[i], deg_sh.at[dst_v.at[i]], add=True)
        return 0

    lax.fori_loop(0, CPT, sa, 0)
    plsc.subcore_barrier()

    @pl.when(c == 0)
    def _():
        pltpu.sync_copy(deg_sh.at[pl.ds(s * RPT, RPT)],
                        deg0_hbm.at[pl.ds(s * RPT, RPT)])

    @pl.when(c == 1)
    def _():
        pltpu.sync_copy(deg_sh.at[pl.ds(s * RPT, RPT)],
                        deg1_hbm.at[pl.ds(s * RPT, RPT)])


@functools.partial(
    pl.kernel,
    out_type=jax.ShapeDtypeStruct((NC, N_PAD, D), jnp.float32),
    mesh=_mesh,
    scratch_types=[
        pltpu.VMEM((PKR, 128), jnp.int32),
        pltpu.VMEM((PKR // 2, 128), jnp.int32),
        pltpu.VMEM((4, KS), jnp.int32),
        pltpu.VMEM((4, KS), jnp.int32),
        pltpu.VMEM((KS, D), jnp.float32),
        pltpu.VMEM((KS, D), jnp.float32),
        pltpu.VMEM((KS, D), jnp.float32),
        pltpu.VMEM((KS, D), jnp.float32),
        pltpu.VMEM_SHARED((N_PAD, D), jnp.float32),
        pltpu.SemaphoreType.DMA,
        pltpu.SemaphoreType.DMA,
        pltpu.SemaphoreType.DMA,
        pltpu.SemaphoreType.DMA,
        pltpu.SemaphoreType.DMA,
        pltpu.SemaphoreType.DMA,
        pltpu.SemaphoreType.DMA,
        pltpu.SemaphoreType.DMA,
    ],
)
def _sc_spmm(y_hbm, pk_hbm, ew_hbm, out_hbm,
             pk_v, ew_v, sidx, didx, rows0, rows1, rows2, rows3, acc_sh,
             gs0, gs1, gs2, gs3, ss0, ss1, ss2, ss3):
    c = lax.axis_index("c")
    s = lax.axis_index("s")
    w = s * NC + c
    rows = (rows0, rows1, rows2, rows3)
    gsem = (gs0, gs1, gs2, gs3)
    ssem = (ss0, ss1, ss2, ss3)
    NCH = 2 * PKR

    def zfill(i, _):
        for j in range(D // 16):
            rows0[i, pl.ds(j * 16, 16)] = jnp.zeros((16,), jnp.float32)
        return 0

    lax.fori_loop(0, KS, zfill, 0)

    def zcopy(i, _):
        pltpu.sync_copy(rows0, acc_sh.at[pl.ds(s * RPT + i * KS, KS)])
        return 0

    lax.fori_loop(0, RPT // KS, zcopy, 0)

    pltpu.sync_copy(pk_hbm.at[w], pk_v)
    pltpu.sync_copy(ew_hbm.at[w], ew_v)
    plsc.subcore_barrier()

    def unpack(t, u):
        r = lax.shift_right_logical(t, 1)
        h = t & 1
        for j in range(KS // 16):
            v = pk_v[r, pl.ds(h * KS + j * 16, 16)]
            sl = pl.ds(j * 16, 16)
            sidx[u, sl] = v & 0xFFFF
            didx[u, sl] = lax.shift_right_logical(v, 16)

    def gather(u):
        pltpu.async_copy(y_hbm.at[sidx.at[u]], rows[u], gsem[u])

    def wait_gather(u):
        pltpu.make_async_copy(y_hbm.at[sidx.at[u]], rows[u], gsem[u]).wait()

    def scatter(u):
        pltpu.async_copy(rows[u], acc_sh.at[didx.at[u]], ssem[u], add=True)

    def wait_scatter(u):
        pltpu.make_async_copy(rows[u], acc_sh.at[didx.at[u]], ssem[u]).wait()

    def compute(t, u):
        r = lax.shift_right_logical(t, 1)
        h = t & 1
        pr = lax.shift_right_logical(t, 2)
        cb = (r & 1) * 64

        def grp(g, _):
            wv = ew_v[pr, pl.ds(cb + g * 16, 16)]
            clo = lax.bitcast_convert_type(lax.shift_left(wv, 16),
                                           jnp.float32)
            chi = lax.bitcast_convert_type(wv & jnp.int32(-65536),
                                           jnp.float32)
            cvec = jnp.where(h == 0, clo, chi)
            for k in range(16):
                cf = jnp.broadcast_to(cvec[k:k + 1], (16,))
                rr = g * 16 + k
                for j in range(D // 16):
                    sl = pl.ds(j * 16, 16)
                    rows[u][rr, sl] = rows[u][rr, sl] * cf
            return 0

        lax.fori_loop(0, KS // 16, grp, 0)
        scatter(u)

    unpack(0, 0)
    gather(0)
    unpack(1, 1)
    gather(1)
    unpack(2, 2)
    gather(2)

    def quad(g, _):
        for u in range(4):
            t = 4 * g + u
            tp = t + 3
            up = (u + 3) % 4

            @pl.when(tp < NCH)
            def _():
                @pl.when(tp >= 4)
                def _():
                    wait_scatter(up)

                unpack(tp, up)
                gather(up)

            wait_gather(u)
            compute(t, u)
        return 0

    lax.fori_loop(0, NCH // 4, quad, 0)

    wait_scatter(0)
    wait_scatter(1)
    wait_scatter(2)
    wait_scatter(3)
    plsc.subcore_barrier()
    pltpu.sync_copy(acc_sh.at[pl.ds(s * RPT, RPT)],
                    out_hbm.at[c, pl.ds(s * RPT, RPT)])


BK = 1024
NBLK = N_PAD // BK


def _tc_dinv_y0_body(deg0_ref, deg1_ref, x_ref, dinv_ref, y0_ref):
    deg = 1.0 + deg0_ref[...] + deg1_ref[...]
    dinv = lax.rsqrt(deg)
    dinv_ref[...] = dinv
    y0_ref[...] = dinv * x_ref[...]


def _tc_layer_body(p0_ref, p1_ref, y_ref, dinv_ref, wt_ref, b_ref, yout_ref):
    agg = (p0_ref[...] + p1_ref[...] + y_ref[...]) * dinv_ref[...]
    h = jnp.maximum(
        jax.lax.dot(agg, wt_ref[...], precision=lax.Precision.HIGHEST,
                    preferred_element_type=jnp.float32) + b_ref[...],
        0.0)
    yout_ref[...] = h * dinv_ref[...]


def _tc_final_body(p0_ref, p1_ref, y_ref, dinv_ref, batch_ref,
                   w2t_ref, b2_ref, u_ref, wm1g_ref, wm1u_ref, bm1_ref,
                   wm2_ref, bm2_ref, out_ref, s_acc, cnt_acc):
    i = pl.program_id(0)

    @pl.when(i == 0)
    def _():
        s_acc[...] = jnp.zeros_like(s_acc)
        cnt_acc[...] = jnp.zeros_like(cnt_acc)

    agg = (p0_ref[...] + p1_ref[...] + y_ref[...]) * dinv_ref[...]
    t2 = jnp.maximum(
        jax.lax.dot(agg, w2t_ref[...], precision=lax.Precision.HIGHEST,
                    preferred_element_type=jnp.float32) + b2_ref[...],
        0.0)
    segid = lax.broadcasted_iota(jnp.int32, (BK, B), 1)
    onehot = (batch_ref[...] == segid).astype(jnp.float32)
    dn = (((0,), (0,)), ((), ()))
    s_acc[...] += lax.dot_general(onehot, t2, dn,
                                  precision=lax.Precision.HIGHEST,
                                  preferred_element_type=jnp.float32)
    cnt_acc[...] += lax.dot_general(onehot, jnp.ones((BK, 1), jnp.float32), dn,
                                    precision=lax.Precision.HIGHEST,
                                    preferred_element_type=jnp.float32)

    @pl.when(i == NBLK - 1)
    def _():
        g = s_acc[...] / jnp.maximum(cnt_acc[...], 1.0)
        z1 = jnp.maximum(
            jax.lax.dot(g, wm1g_ref[...], precision=lax.Precision.HIGHEST,
                        preferred_element_type=jnp.float32)
            + jax.lax.dot(u_ref[...], wm1u_ref[...],
                          precision=lax.Precision.HIGHEST,
                          preferred_element_type=jnp.float32)
            + bm1_ref[...],
            0.0)
        out_ref[...] = (jax.lax.dot(z1, wm2_ref[...],
                                    precision=lax.Precision.HIGHEST,
                                    preferred_element_type=jnp.float32)
                        + bm2_ref[...])


def _rows_spec(cols):
    return pl.BlockSpec((BK, cols), lambda i: (i, 0))


def _full_spec(shape):
    return pl.BlockSpec(shape, lambda i: tuple(0 for _ in shape))


def kernel(x, edge_index, edge_attr, batch, u, W1, b1, W2, b2, Wm1, bm1, Wm2, bm2):
    f32 = jnp.float32
    x = x.astype(f32)
    x_pad = jnp.pad(x, ((0, N_PAD - N), (0, 0)))
    d2 = edge_attr.reshape(NW, CPT, K).astype(f32)
    src2 = edge_index[0].reshape(NW, CPT, K)
    dst2 = edge_index[1].reshape(NW, CPT, K)
    batch_pad = jnp.pad(batch, (0, N_PAD - N), constant_values=B).reshape(N_PAD, 1)
    W1t = W1.T.astype(f32)
    W2t = W2.T.astype(f32)
    Wm1t = Wm1.T.astype(f32)
    Wm1g = Wm1t[:H]
    Wm1u = Wm1t[H:]
    b1r = b1.reshape(1, H).astype(f32)
    b2r = b2.reshape(1, H).astype(f32)
    bm1r = bm1.reshape(1, 128).astype(f32)
    Wm2t = Wm2.T.astype(f32)
    bm2r = bm2.reshape(1, 1).astype(f32)

    ept_raw = E // NW
    srcw = edge_index[0].reshape(NW, ept_raw)
    dstw = edge_index[1].reshape(NW, ept_raw)
    pad_src = jnp.broadcast_to(jnp.arange(EPAD, dtype=jnp.int32)[None, :],
                               (NW, EPAD))
    pad_dst = pad_src + N
    pkw = jnp.concatenate([srcw | (dstw << 16),
                           pad_src | (pad_dst << 16)], axis=1)
    pkw = pkw.reshape(NW, PKR, 128)

    ew2, dga, dgb = _sc_ew_deg(d2, dst2)
    deg0 = dga.reshape(N_PAD, 1)
    deg1 = dgb.reshape(N_PAD, 1)
    ew_sp = jnp.concatenate([ew2.reshape(NW, ept_raw),
                             jnp.zeros((NW, EPAD), jnp.float32)], axis=1)
    ew_sp = ew_sp.reshape(NW, PKR, 128)
    bl = jax.lax.bitcast_convert_type(ew_sp[..., :64], jnp.int32)
    bh = jax.lax.bitcast_convert_type(ew_sp[..., 64:], jnp.int32)
    ewb = (jax.lax.shift_right_logical(bl + 0x8000, 16)
           | ((bh + 0x8000) & jnp.int32(-65536)))
    ewb = ewb.reshape(NW, PKR // 2, 128)

    dinv, y0 = pl.pallas_call(
        _tc_dinv_y0_body,
        grid=(NBLK,),
        in_specs=[_rows_spec(1), _rows_spec(1), _rows_spec(D)],
        out_specs=[_rows_spec(1), _rows_spec(D)],
        out_shape=[
            jax.ShapeDtypeStruct((N_PAD, 1), f32),
            jax.ShapeDtypeStruct((N_PAD, D), f32),
        ],
    )(deg0, deg1, x_pad)

    p = _sc_spmm(y0, pkw, ewb)

    y1 = pl.pallas_call(
        _tc_layer_body,
        grid=(NBLK,),
        in_specs=[_rows_spec(D), _rows_spec(D), _rows_spec(D), _rows_spec(1),
                  _full_spec((D, H)), _full_spec((1, H))],
        out_specs=_rows_spec(H),
        out_shape=jax.ShapeDtypeStruct((N_PAD, H), f32),
    )(p[0], p[1], y0, dinv, W1t, b1r)

    q = _sc_spmm(y1, pkw, ewb)

    out = pl.pallas_call(
        _tc_final_body,
        grid=(NBLK,),
        in_specs=[_rows_spec(H), _rows_spec(H), _rows_spec(H), _rows_spec(1),
                  _rows_spec(1),
                  _full_spec((H, H)), _full_spec((1, H)), _full_spec((B, G)),
                  _full_spec((H, 128)), _full_spec((G, 128)),
                  _full_spec((1, 128)), _full_spec((128, 1)),
                  _full_spec((1, 1))],
        out_specs=pl.BlockSpec((B, 1), lambda i: (0, 0)),
        out_shape=jax.ShapeDtypeStruct((B, 1), f32),
        scratch_shapes=[
            pltpu.VMEM((B, H), f32),
            pltpu.VMEM((B, 1), f32),
        ],
    )(q[0], q[1], y1, dinv, batch_pad,
      W2t, b2r, u, Wm1g, Wm1u, bm1r, Wm2t, bm2r)

    return out.reshape(-1)


# speedup vs baseline: 25.6846x; 1.0137x over previous
import functools

import jax
import jax.numpy as jnp
from jax import lax
from jax.experimental import pallas as pl
from jax.experimental.pallas import tpu as pltpu
from jax.experimental.pallas import tpu_sc as plsc

N = 10000
E = 320000
D = 128
H = 128
G = 32
B = 16

N_PAD = 10240
NC = 2
NS = 16
NW = NC * NS
RPT = N_PAD // NS

K = 80
CPT = (E // K) // NW

KS = 64
PKR = 80
EPT = PKR * 128
EPAD = EPT - E // NW

_mesh = plsc.VectorSubcoreMesh(core_axis_name="c", subcore_axis_name="s")


@functools.partial(
    pl.kernel,
    out_type=(
        jax.ShapeDtypeStruct((NW, CPT, K), jnp.float32),
        jax.ShapeDtypeStruct((N_PAD,), jnp.float32),
        jax.ShapeDtypeStruct((N_PAD,), jnp.float32),
    ),
    mesh=_mesh,
    scratch_types=[
        pltpu.VMEM((CPT, K), jnp.float32),
        pltpu.VMEM((CPT, K), jnp.float32),
        pltpu.VMEM((CPT, K), jnp.int32),
        pltpu.VMEM((RPT,), jnp.float32),
        pltpu.VMEM_SHARED((N_PAD,), jnp.float32),
    ],
)
def _sc_ew_deg(d_hbm, dst_hbm, ew_hbm, deg0_hbm, deg1_hbm,
               d_v, ew_v, dst_v, zero_v, deg_sh):
    c = lax.axis_index("c")
    s = lax.axis_index("s")
    w = s * NC + c

    def zfill(i, _):
        zero_v[pl.ds(i * 16, 16)] = jnp.zeros((16,), jnp.float32)
        return 0

    lax.fori_loop(0, RPT // 16, zfill, 0)
    pltpu.sync_copy(zero_v, deg_sh.at[pl.ds(s * RPT, RPT)])

    pltpu.sync_copy(d_hbm.at[w], d_v)
    pltpu.sync_copy(dst_hbm.at[w], dst_v)

    def cw(i, _):
        for j in range(K // 16):
            sl = pl.ds(j * 16, 16)
            d = d_v[i, sl]
            t = d * 0.2
            ew_v[i, sl] = jnp.exp(-(t * t))
        return 0

    lax.fori_loop(0, CPT, cw, 0)
    pltpu.sync_copy(ew_v, ew_hbm.at[w])

    plsc.subcore_barrier()

    def sa(i, _):
        pltpu.sync_copy(ew_v.at[i], deg_sh.at[dst_v.at[i]], add=True)
        return 0

    lax.fori_loop(0, CPT, sa, 0)
    plsc.subcore_barrier()

    @pl.when(c == 0)
    def _():
        pltpu.sync_copy(deg_sh.at[pl.ds(s * RPT, RPT)],
                        deg0_hbm.at[pl.ds(s * RPT, RPT)])

    @pl.when(c == 1)
    def _():
        pltpu.sync_copy(deg_sh.at[pl.ds(s * RPT, RPT)],
                        deg1_hbm.at[pl.ds(s * RPT, RPT)])


@functools.partial(
    pl.kernel,
    out_type=jax.ShapeDtypeStruct((NC, N_PAD, D), jnp.float32),
    mesh=_mesh,
    scratch_types=[
        pltpu.VMEM((PKR, 128), jnp.int32),
        pltpu.VMEM((PKR, 128), jnp.float32),
        pltpu.VMEM((3, KS), jnp.int32),
        pltpu.VMEM((3, KS), jnp.int32),
        pltpu.VMEM((KS, D), jnp.float32),
        pltpu.VMEM((KS, D), jnp.float32),
        pltpu.VMEM((KS, D), jnp.float32),
        pltpu.VMEM_SHARED((N_PAD, D), jnp.float32),
        pltpu.SemaphoreType.DMA,
        pltpu.SemaphoreType.DMA,
        pltpu.SemaphoreType.DMA,
        pltpu.SemaphoreType.DMA,
        pltpu.SemaphoreType.DMA,
        pltpu.SemaphoreType.DMA,
    ],
)
def _sc_spmm(y_hbm, pk_hbm, ew_hbm, out_hbm,
             pk_v, ew_v, sidx, didx, rows0, rows1, rows2, acc_sh,
             gs0, gs1, gs2, ss0, ss1, ss2):
    c = lax.axis_index("c")
    s = lax.axis_index("s")
    w = s * NC + c
    rows = (rows0, rows1, rows2)
    gsem = (gs0, gs1, gs2)
    ssem = (ss0, ss1, ss2)
    NCH = 2 * PKR

    def zfill(i, _):
        for j in range(D // 16):
            rows0[i, pl.ds(j * 16, 16)] = jnp.zeros((16,), jnp.float32)
        return 0

    lax.fori_loop(0, KS, zfill, 0)

    def zcopy(i, _):
        pltpu.sync_copy(rows0, acc_sh.at[pl.ds(s * RPT + i * KS, KS)])
        return 0

    lax.fori_loop(0, RPT // KS, zcopy, 0)

    pltpu.sync_copy(pk_hbm.at[w], pk_v)
    pltpu.sync_copy(ew_hbm.at[w], ew_v)
    plsc.subcore_barrier()

    def unpack(t, u):
        r = lax.shift_right_logical(t, 1)
        h = t & 1
        for j in range(KS // 16):
            v = pk_v[r, pl.ds(h * KS + j * 16, 16)]
            sl = pl.ds(j * 16, 16)
            sidx[u, sl] = v & 0xFFFF
            didx[u, sl] = lax.shift_right_logical(v, 16)

    def gather(u):
        pltpu.async_copy(y_hbm.at[sidx.at[u]], rows[u], gsem[u])

    def wait_gather(u):
        pltpu.make_async_copy(y_hbm.at[sidx.at[u]], rows[u], gsem[u]).wait()

    def scatter(u):
        pltpu.async_copy(rows[u], acc_sh.at[didx.at[u]], ssem[u], add=True)

    def wait_scatter(u):
        pltpu.make_async_copy(rows[u], acc_sh.at[didx.at[u]], ssem[u]).wait()

    def compute(t, u):
        r = lax.shift_right_logical(t, 1)
        h = t & 1

        def grp(g, _):
            cvec = ew_v[r, pl.ds(h * KS + g * 16, 16)]
            for k in range(16):
                cf = jnp.broadcast_to(cvec[k:k + 1], (16,))
                rr = g * 16 + k
                for j in range(D // 16):
                    sl = pl.ds(j * 16, 16)
                    rows[u][rr, sl] = rows[u][rr, sl] * cf
            return 0

        lax.fori_loop(0, KS // 16, grp, 0)
        scatter(u)

    unpack(0, 0)
    gather(0)
    unpack(1, 1)
    gather(1)

    def tri(g, _):
        for u in range(3):
            t = 3 * g + u
            tp = t + 2
            up = (u + 2) % 3

            @pl.when(tp < NCH)
            def _():
                @pl.when(tp >= 3)
                def _():
                    wait_scatter(up)

                unpack(tp, up)
                gather(up)

            wait_gather(u)
            compute(t, u)
        return 0

    lax.fori_loop(0, (NCH - 1) // 3, tri, 0)
    wait_gather(0)
    compute(NCH - 1, 0)

    wait_scatter(0)
    wait_scatter(1)
    wait_scatter(2)
    plsc.subcore_barrier()
    pltpu.sync_copy(acc_sh.at[pl.ds(s * RPT, RPT)],
                    out_hbm.at[c, pl.ds(s * RPT, RPT)])


BK = 1024
NBLK = N_PAD // BK


def _tc_dinv_y0_body(deg0_ref, deg1_ref, x_ref, dinv_ref, y0_ref):
    deg = 1.0 + deg0_ref[...] + deg1_ref[...]
    dinv = lax.rsqrt(deg)
    dinv_ref[...] = dinv
    y0_ref[...] = dinv * x_ref[...]


def _tc_layer_body(p0_ref, p1_ref, y_ref, dinv_ref, wt_ref, b_ref, yout_ref):
    agg = (p0_ref[...] + p1_ref[...] + y_ref[...]) * dinv_ref[...]
    h = jnp.maximum(
        jax.lax.dot(agg, wt_ref[...], precision=lax.Precision.HIGHEST,
                    preferred_element_type=jnp.float32) + b_ref[...],
        0.0)
    yout_ref[...] = h * dinv_ref[...]


def _tc_final_body(p0_ref, p1_ref, y_ref, dinv_ref, batch_ref,
                   w2t_ref, b2_ref, u_ref, wm1g_ref, wm1u_ref, bm1_ref,
                   wm2_ref, bm2_ref, out_ref, s_acc, cnt_acc):
    i = pl.program_id(0)

    @pl.when(i == 0)
    def _():
        s_acc[...] = jnp.zeros_like(s_acc)
        cnt_acc[...] = jnp.zeros_like(cnt_acc)

    agg = (p0_ref[...] + p1_ref[...] + y_ref[...]) * dinv_ref[...]
    t2 = jnp.maximum(
        jax.lax.dot(agg, w2t_ref[...], precision=lax.Precision.HIGHEST,
                    preferred_element_type=jnp.float32) + b2_ref[...],
        0.0)
    segid = lax.broadcasted_iota(jnp.int32, (BK, B), 1)
    onehot = (batch_ref[...] == segid).astype(jnp.float32)
    dn = (((0,), (0,)), ((), ()))
    s_acc[...] += lax.dot_general(onehot, t2, dn,
                                  precision=lax.Precision.HIGHEST,
                                  preferred_element_type=jnp.float32)
    cnt_acc[...] += lax.dot_general(onehot, jnp.ones((BK, 1), jnp.float32), dn,
                                    precision=lax.Precision.HIGHEST,
                                    preferred_element_type=jnp.float32)

    @pl.when(i == NBLK - 1)
    def _():
        g = s_acc[...] / jnp.maximum(cnt_acc[...], 1.0)
        z1 = jnp.maximum(
            jax.lax.dot(g, wm1g_ref[...], precision=lax.Precision.HIGHEST,
                        preferred_element_type=jnp.float32)
            + jax.lax.dot(u_ref[...], wm1u_ref[...],
                          precision=lax.Precision.HIGHEST,
                          preferred_element_type=jnp.float32)
            + bm1_ref[...],
            0.0)
        out_ref[...] = (jax.lax.dot(z1, wm2_ref[...],
                                    precision=lax.Precision.HIGHEST,
                                    preferred_element_type=jnp.float32)
                        + bm2_ref[...])


def _rows_spec(cols):
    return pl.BlockSpec((BK, cols), lambda i: (i, 0))


def _full_spec(shape):
    return pl.BlockSpec(shape, lambda i: tuple(0 for _ in shape))


def kernel(x, edge_index, edge_attr, batch, u, W1, b1, W2, b2, Wm1, bm1, Wm2, bm2):
    f32 = jnp.float32
    x = x.astype(f32)
    x_pad = jnp.pad(x, ((0, N_PAD - N), (0, 0)))
    d2 = edge_attr.reshape(NW, CPT, K).astype(f32)
    src2 = edge_index[0].reshape(NW, CPT, K)
    dst2 = edge_index[1].reshape(NW, CPT, K)
    batch_pad = jnp.pad(batch, (0, N_PAD - N), constant_values=B).reshape(N_PAD, 1)
    W1t = W1.T.astype(f32)
    W2t = W2.T.astype(f32)
    Wm1t = Wm1.T.astype(f32)
    Wm1g = Wm1t[:H]
    Wm1u = Wm1t[H:]
    b1r = b1.reshape(1, H).astype(f32)
    b2r = b2.reshape(1, H).astype(f32)
    bm1r = bm1.reshape(1, 128).astype(f32)
    Wm2t = Wm2.T.astype(f32)
    bm2r = bm2.reshape(1, 1).astype(f32)

    ept_raw = E // NW
    srcw = edge_index[0].reshape(NW, ept_raw)
    dstw = edge_index[1].reshape(NW, ept_raw)
    pad_src = jnp.broadcast_to(jnp.arange(EPAD, dtype=jnp.int32)[None, :],
                               (NW, EPAD))
    pad_dst = pad_src + N
    pkw = jnp.concatenate([srcw | (dstw << 16),
                           pad_src | (pad_dst << 16)], axis=1)
    pkw = pkw.reshape(NW, PKR, 128)

    ew2, dga, dgb = _sc_ew_deg(d2, dst2)
    deg0 = dga.reshape(N_PAD, 1)
    deg1 = dgb.reshape(N_PAD, 1)
    ew_sp = jnp.concatenate([ew2.reshape(NW, ept_raw),
                             jnp.zeros((NW, EPAD), jnp.float32)], axis=1)
    ew_sp = ew_sp.reshape(NW, PKR, 128)

    dinv, y0 = pl.pallas_call(
        _tc_dinv_y0_body,
        grid=(NBLK,),
        in_specs=[_rows_spec(1), _rows_spec(1), _rows_spec(D)],
        out_specs=[_rows_spec(1), _rows_spec(D)],
        out_shape=[
            jax.ShapeDtypeStruct((N_PAD, 1), f32),
            jax.ShapeDtypeStruct((N_PAD, D), f32),
        ],
    )(deg0, deg1, x_pad)

    p = _sc_spmm(y0, pkw, ew_sp)

    y1 = pl.pallas_call(
        _tc_layer_body,
        grid=(NBLK,),
        in_specs=[_rows_spec(D), _rows_spec(D), _rows_spec(D), _rows_spec(1),
                  _full_spec((D, H)), _full_spec((1, H))],
        out_specs=_rows_spec(H),
        out_shape=jax.ShapeDtypeStruct((N_PAD, H), f32),
    )(p[0], p[1], y0, dinv, W1t, b1r)

    q = _sc_spmm(y1, pkw, ew_sp)

    out = pl.pallas_call(
        _tc_final_body,
        grid=(NBLK,),
        in_specs=[_rows_spec(H), _rows_spec(H), _rows_spec(H), _rows_spec(1),
                  _rows_spec(1),
                  _full_spec((H, H)), _full_spec((1, H)), _full_spec((B, G)),
                  _full_spec((H, 128)), _full_spec((G, 128)),
                  _full_spec((1, 128)), _full_spec((128, 1)),
                  _full_spec((1, 1))],
        out_specs=pl.BlockSpec((B, 1), lambda i: (0, 0)),
        out_shape=jax.ShapeDtypeStruct((B, 1), f32),
        scratch_shapes=[
            pltpu.VMEM((B, H), f32),
            pltpu.VMEM((B, 1), f32),
        ],
    )(q[0], q[1], y1, dinv, batch_pad,
      W2t, b2r, u, Wm1g, Wm1u, bm1r, Wm2t, bm2r)

    return out.reshape(-1)


# speedup vs baseline: 25.8640x; 1.0070x over previous
import functools

import jax
import jax.numpy as jnp
from jax import lax
from jax.experimental import pallas as pl
from jax.experimental.pallas import tpu as pltpu
from jax.experimental.pallas import tpu_sc as plsc

N = 10000
E = 320000
D = 128
H = 128
G = 32
B = 16

N_PAD = 10240
NC = 2
NS = 16
NW = NC * NS
RPT = N_PAD // NS

K = 80
CPT = (E // K) // NW

KS = 64
PKR = 80
EPT = PKR * 128
EPAD = EPT - E // NW

_mesh = plsc.VectorSubcoreMesh(core_axis_name="c", subcore_axis_name="s")


@functools.partial(
    pl.kernel,
    out_type=(
        jax.ShapeDtypeStruct((NW, CPT, K), jnp.float32),
        jax.ShapeDtypeStruct((N_PAD,), jnp.float32),
        jax.ShapeDtypeStruct((N_PAD,), jnp.float32),
    ),
    mesh=_mesh,
    scratch_types=[
        pltpu.VMEM((CPT, K), jnp.float32),
        pltpu.VMEM((CPT, K), jnp.float32),
        pltpu.VMEM((CPT, K), jnp.int32),
        pltpu.VMEM((RPT,), jnp.float32),
        pltpu.VMEM_SHARED((N_PAD,), jnp.float32),
        pltpu.SemaphoreType.DMA,
    ],
)
def _sc_ew_deg(d_hbm, dst_hbm, ew_hbm, deg0_hbm, deg1_hbm,
               d_v, ew_v, dst_v, zero_v, deg_sh, dsem):
    c = lax.axis_index("c")
    s = lax.axis_index("s")
    w = s * NC + c

    def zfill(i, _):
        zero_v[pl.ds(i * 16, 16)] = jnp.zeros((16,), jnp.float32)
        return 0

    lax.fori_loop(0, RPT // 16, zfill, 0)
    pltpu.sync_copy(zero_v, deg_sh.at[pl.ds(s * RPT, RPT)])

    pltpu.sync_copy(d_hbm.at[w], d_v)
    pltpu.sync_copy(dst_hbm.at[w], dst_v)

    def cw(i, _):
        for j in range(K // 16):
            sl = pl.ds(j * 16, 16)
            d = d_v[i, sl]
            t = d * 0.2
            ew_v[i, sl] = jnp.exp(-(t * t))
        return 0

    lax.fori_loop(0, CPT, cw, 0)
    pltpu.sync_copy(ew_v, ew_hbm.at[w])

    plsc.subcore_barrier()

    def sa(i, _):
        pltpu.async_copy(ew_v.at[i], deg_sh.at[dst_v.at[i]], dsem, add=True)
        return 0

    lax.fori_loop(0, CPT, sa, 0)

    def sw(i, _):
        pltpu.make_async_copy(ew_v.at[i], deg_sh.at[dst_v.at[i]],
                              dsem).wait()
        return 0

    lax.fori_loop(0, CPT, sw, 0)
    plsc.subcore_barrier()

    @pl.when(c == 0)
    def _():
        pltpu.sync_copy(deg_sh.at[pl.ds(s * RPT, RPT)],
                        deg0_hbm.at[pl.ds(s * RPT, RPT)])

    @pl.when(c == 1)
    def _():
        pltpu.sync_copy(deg_sh.at[pl.ds(s * RPT, RPT)],
                        deg1_hbm.at[pl.ds(s * RPT, RPT)])


@functools.partial(
    pl.kernel,
    out_type=jax.ShapeDtypeStruct((NC, N_PAD, D), jnp.float32),
    mesh=_mesh,
    scratch_types=[
        pltpu.VMEM((PKR, 128), jnp.int32),
        pltpu.VMEM((PKR, 128), jnp.float32),
        pltpu.VMEM((3, KS), jnp.int32),
        pltpu.VMEM((3, KS), jnp.int32),
        pltpu.VMEM((KS, D), jnp.float32),
        pltpu.VMEM((KS, D), jnp.float32),
        pltpu.VMEM((KS, D), jnp.float32),
        pltpu.VMEM_SHARED((N_PAD, D), jnp.float32),
        pltpu.SemaphoreType.DMA,
        pltpu.SemaphoreType.DMA,
        pltpu.SemaphoreType.DMA,
        pltpu.SemaphoreType.DMA,
        pltpu.SemaphoreType.DMA,
        pltpu.SemaphoreType.DMA,
    ],
)
def _sc_spmm(y_hbm, pk_hbm, ew_hbm, out_hbm,
             pk_v, ew_v, sidx, didx, rows0, rows1, rows2, acc_sh,
             gs0, gs1, gs2, ss0, ss1, ss2):
    c = lax.axis_index("c")
    s = lax.axis_index("s")
    w = s * NC + c
    rows = (rows0, rows1, rows2)
    gsem = (gs0, gs1, gs2)
    ssem = (ss0, ss1, ss2)
    NCH = 2 * PKR

    def zfill(i, _):
        for j in range(D // 16):
            rows0[i, pl.ds(j * 16, 16)] = jnp.zeros((16,), jnp.float32)
        return 0

    lax.fori_loop(0, KS, zfill, 0)

    def zcopy(i, _):
        pltpu.sync_copy(rows0, acc_sh.at[pl.ds(s * RPT + i * KS, KS)])
        return 0

    lax.fori_loop(0, RPT // KS, zcopy, 0)

    pltpu.sync_copy(pk_hbm.at[w], pk_v)
    pltpu.sync_copy(ew_hbm.at[w], ew_v)
    plsc.subcore_barrier()

    def unpack(t, u):
        r = lax.shift_right_logical(t, 1)
        h = t & 1
        for j in range(KS // 16):
            v = pk_v[r, pl.ds(h * KS + j * 16, 16)]
            sl = pl.ds(j * 16, 16)
            sidx[u, sl] = v & 0xFFFF
            didx[u, sl] = lax.shift_right_logical(v, 16)

    def gather(u):
        pltpu.async_copy(y_hbm.at[sidx.at[u]], rows[u], gsem[u])

    def wait_gather(u):
        pltpu.make_async_copy(y_hbm.at[sidx.at[u]], rows[u], gsem[u]).wait()

    def scatter(u):
        pltpu.async_copy(rows[u], acc_sh.at[didx.at[u]], ssem[u], add=True)

    def wait_scatter(u):
        pltpu.make_async_copy(rows[u], acc_sh.at[didx.at[u]], ssem[u]).wait()

    def compute(t, u):
        r = lax.shift_right_logical(t, 1)
        h = t & 1

        def grp(g, _):
            cvec = ew_v[r, pl.ds(h * KS + g * 16, 16)]
            for k in range(16):
                cf = jnp.broadcast_to(cvec[k:k + 1], (16,))
                rr = g * 16 + k
                for j in range(D // 16):
                    sl = pl.ds(j * 16, 16)
                    rows[u][rr, sl] = rows[u][rr, sl] * cf
            return 0

        lax.fori_loop(0, KS // 16, grp, 0)
        scatter(u)

    unpack(0, 0)
    gather(0)
    unpack(1, 1)
    gather(1)

    def tri(g, _):
        for u in range(3):
            t = 3 * g + u
            tp = t + 2
            up = (u + 2) % 3

            @pl.when(tp < NCH)
            def _():
                @pl.when(tp >= 3)
                def _():
                    wait_scatter(up)

                unpack(tp, up)
                gather(up)

            wait_gather(u)
            compute(t, u)
        return 0

    lax.fori_loop(0, (NCH - 1) // 3, tri, 0)
    wait_gather(0)
    compute(NCH - 1, 0)

    wait_scatter(0)
    wait_scatter(1)
    wait_scatter(2)
    plsc.subcore_barrier()
    pltpu.sync_copy(acc_sh.at[pl.ds(s * RPT, RPT)],
                    out_hbm.at[c, pl.ds(s * RPT, RPT)])


BK = 1024
NBLK = N_PAD // BK


def _tc_dinv_y0_body(deg0_ref, deg1_ref, x_ref, dinv_ref, y0_ref):
    deg = 1.0 + deg0_ref[...] + deg1_ref[...]
    dinv = lax.rsqrt(deg)
    dinv_ref[...] = dinv
    y0_ref[...] = dinv * x_ref[...]


def _tc_layer_body(p0_ref, p1_ref, y_ref, dinv_ref, wt_ref, b_ref, yout_ref):
    agg = (p0_ref[...] + p1_ref[...] + y_ref[...]) * dinv_ref[...]
    h = jnp.maximum(
        jax.lax.dot(agg, wt_ref[...], precision=lax.Precision.HIGHEST,
                    preferred_element_type=jnp.float32) + b_ref[...],
        0.0)
    yout_ref[...] = h * dinv_ref[...]


def _tc_final_body(p0_ref, p1_ref, y_ref, dinv_ref, batch_ref,
                   w2t_ref, b2_ref, u_ref, wm1g_ref, wm1u_ref, bm1_ref,
                   wm2_ref, bm2_ref, out_ref, s_acc, cnt_acc):
    i = pl.program_id(0)

    @pl.when(i == 0)
    def _():
        s_acc[...] = jnp.zeros_like(s_acc)
        cnt_acc[...] = jnp.zeros_like(cnt_acc)

    agg = (p0_ref[...] + p1_ref[...] + y_ref[...]) * dinv_ref[...]
    t2 = jnp.maximum(
        jax.lax.dot(agg, w2t_ref[...], precision=lax.Precision.HIGHEST,
                    preferred_element_type=jnp.float32) + b2_ref[...],
        0.0)
    segid = lax.broadcasted_iota(jnp.int32, (BK, B), 1)
    onehot = (batch_ref[...] == segid).astype(jnp.float32)
    dn = (((0,), (0,)), ((), ()))
    s_acc[...] += lax.dot_general(onehot, t2, dn,
                                  precision=lax.Precision.HIGHEST,
                                  preferred_element_type=jnp.float32)
    cnt_acc[...] += lax.dot_general(onehot, jnp.ones((BK, 1), jnp.float32), dn,
                                    precision=lax.Precision.HIGHEST,
                                    preferred_element_type=jnp.float32)

    @pl.when(i == NBLK - 1)
    def _():
        g = s_acc[...] / jnp.maximum(cnt_acc[...], 1.0)
        z1 = jnp.maximum(
            jax.lax.dot(g, wm1g_ref[...], precision=lax.Precision.HIGHEST,
                        preferred_element_type=jnp.float32)
            + jax.lax.dot(u_ref[...], wm1u_ref[...],
                          precision=lax.Precision.HIGHEST,
                          preferred_element_type=jnp.float32)
            + bm1_ref[...],
            0.0)
        out_ref[...] = (jax.lax.dot(z1, wm2_ref[...],
                                    precision=lax.Precision.HIGHEST,
                                    preferred_element_type=jnp.float32)
                        + bm2_ref[...])


def _rows_spec(cols):
    return pl.BlockSpec((BK, cols), lambda i: (i, 0))


def _full_spec(shape):
    return pl.BlockSpec(shape, lambda i: tuple(0 for _ in shape))


def kernel(x, edge_index, edge_attr, batch, u, W1, b1, W2, b2, Wm1, bm1, Wm2, bm2):
    f32 = jnp.float32
    x = x.astype(f32)
    x_pad = jnp.pad(x, ((0, N_PAD - N), (0, 0)))
    d2 = edge_attr.reshape(NW, CPT, K).astype(f32)
    src2 = edge_index[0].reshape(NW, CPT, K)
    dst2 = edge_index[1].reshape(NW, CPT, K)
    batch_pad = jnp.pad(batch, (0, N_PAD - N), constant_values=B).reshape(N_PAD, 1)
    W1t = W1.T.astype(f32)
    W2t = W2.T.astype(f32)
    Wm1t = Wm1.T.astype(f32)
    Wm1g = Wm1t[:H]
    Wm1u = Wm1t[H:]
    b1r = b1.reshape(1, H).astype(f32)
    b2r = b2.reshape(1, H).astype(f32)
    bm1r = bm1.reshape(1, 128).astype(f32)
    Wm2t = Wm2.T.astype(f32)
    bm2r = bm2.reshape(1, 1).astype(f32)

    ept_raw = E // NW
    srcw = edge_index[0].reshape(NW, ept_raw)
    dstw = edge_index[1].reshape(NW, ept_raw)
    pad_src = jnp.broadcast_to(jnp.arange(EPAD, dtype=jnp.int32)[None, :],
                               (NW, EPAD))
    pad_dst = pad_src + N
    pkw = jnp.concatenate([srcw | (dstw << 16),
                           pad_src | (pad_dst << 16)], axis=1)
    pkw = pkw.reshape(NW, PKR, 128)

    ew2, dga, dgb = _sc_ew_deg(d2, dst2)
    deg0 = dga.reshape(N_PAD, 1)
    deg1 = dgb.reshape(N_PAD, 1)
    ew_sp = jnp.concatenate([ew2.reshape(NW, ept_raw),
                             jnp.zeros((NW, EPAD), jnp.float32)], axis=1)
    ew_sp = ew_sp.reshape(NW, PKR, 128)

    dinv, y0 = pl.pallas_call(
        _tc_dinv_y0_body,
        grid=(NBLK,),
        in_specs=[_rows_spec(1), _rows_spec(1), _rows_spec(D)],
        out_specs=[_rows_spec(1), _rows_spec(D)],
        out_shape=[
            jax.ShapeDtypeStruct((N_PAD, 1), f32),
            jax.ShapeDtypeStruct((N_PAD, D), f32),
        ],
    )(deg0, deg1, x_pad)

    p = _sc_spmm(y0, pkw, ew_sp)

    y1 = pl.pallas_call(
        _tc_layer_body,
        grid=(NBLK,),
        in_specs=[_rows_spec(D), _rows_spec(D), _rows_spec(D), _rows_spec(1),
                  _full_spec((D, H)), _full_spec((1, H))],
        out_specs=_rows_spec(H),
        out_shape=jax.ShapeDtypeStruct((N_PAD, H), f32),
    )(p[0], p[1], y0, dinv, W1t, b1r)

    q = _sc_spmm(y1, pkw, ew_sp)

    out = pl.pallas_call(
        _tc_final_body,
        grid=(NBLK,),
        in_specs=[_rows_spec(H), _rows_spec(H), _rows_spec(H), _rows_spec(1),
                  _rows_spec(1),
                  _full_spec((H, H)), _full_spec((1, H)), _full_spec((B, G)),
                  _full_spec((H, 128)), _full_spec((G, 128)),
                  _full_spec((1, 128)), _full_spec((128, 1)),
                  _full_spec((1, 1))],
        out_specs=pl.BlockSpec((B, 1), lambda i: (0, 0)),
        out_shape=jax.ShapeDtypeStruct((B, 1), f32),
        scratch_shapes=[
            pltpu.VMEM((B, H), f32),
            pltpu.VMEM((B, 1), f32),
        ],
    )(q[0], q[1], y1, dinv, batch_pad,
      W2t, b2r, u, Wm1g, Wm1u, bm1r, Wm2t, bm2r)

    return out.reshape(-1)
